# baseline (speedup 1.0000x reference)
import functools

import jax
import jax.numpy as jnp
from jax import lax
from jax.experimental import pallas as pl
from jax.experimental.pallas import tpu as pltpu

TOPK = 16
KB = 4096
INF = float("inf")
FIMAX = float(2**24)


def _topk_body(n_valid, nkb, q_ref, kt_ref, q2_ref, k2_ref, vals_ref, idx_ref,
               dd_ref, rv_ref, ri_ref, cv_ref, ci_ref):
    kb = pl.program_id(0)

    @pl.when(kb == 0)
    def _init():
        rv_ref[...] = jnp.full(rv_ref.shape, INF, jnp.float32)
        ri_ref[...] = jnp.zeros(ri_ref.shape, jnp.float32)
        cv_ref[...] = jnp.full(cv_ref.shape, INF, jnp.float32)
        ci_ref[...] = jnp.full(ci_ref.shape, FIMAX, jnp.float32)

    q = q_ref[...]
    kt = kt_ref[...]
    q2 = q2_ref[...]
    k2 = k2_ref[...]
    qk = jnp.dot(q, kt, preferred_element_type=jnp.float32)
    d2 = q2 + k2 - 2.0 * qk
    d2 = jnp.maximum(d2, 0.0)

    base = (kb * KB).astype(jnp.float32)

    def flane():
        return lax.broadcasted_iota(
            jnp.int32, (dd_ref.shape[0], KB), 1).astype(jnp.float32)

    flocal = flane()
    d2 = jnp.where(flocal < float(n_valid) - base, d2, INF)

    dd = d2
    m = jnp.min(dd, axis=1, keepdims=True)
    sel = jnp.min(jnp.where(dd == m, flocal, FIMAX), axis=1, keepdims=True)
    cv_ref[:, 0:1] = m
    ci_ref[:, 0:1] = sel + base
    for i in range(1, TOPK):
        dd = jnp.where(flane() == sel, INF, dd)
        m = jnp.min(dd, axis=1, keepdims=True)
        sel = jnp.min(jnp.where(dd == m, flane(), FIMAX), axis=1,
                      keepdims=True)
        cv_ref[:, i:i + 1] = m
        ci_ref[:, i:i + 1] = sel + base

    def _merge():
        cv = jnp.concatenate([rv_ref[...], cv_ref[...]], axis=1)
        ci = jnp.concatenate([ri_ref[...], ci_ref[...]], axis=1)
        nv, ni = [], []
        for _ in range(TOPK):
            m = jnp.min(cv, axis=1, keepdims=True)
            sel = jnp.min(jnp.where(cv == m, ci, FIMAX), axis=1, keepdims=True)
            nv.append(m)
            ni.append(sel)
            cv = jnp.where(ci == sel, INF, cv)
        rv_ref[...] = jnp.concatenate(nv, axis=1)
        ri_ref[...] = jnp.concatenate(ni, axis=1)

    _merge()

    @pl.when(kb == nkb - 1)
    def _finalize():
        vals_ref[...] = rv_ref[...]
        idx_ref[...] = ri_ref[...].astype(jnp.int32)


def kernel(queries, keys):
    Q, D = queries.shape
    K = keys.shape[0]
    nkb = pl.cdiv(K, KB)
    kpad = nkb * KB

    q2 = jnp.sum(queries * queries, axis=1, keepdims=True)
    k2 = jnp.sum(keys * keys, axis=1)[None, :]
    keys_t = jnp.pad(keys.T, ((0, 0), (0, kpad - K)))
    k2p = jnp.pad(k2, ((0, 0), (0, kpad - K)))

    vals, idx = pl.pallas_call(
        functools.partial(_topk_body, K, nkb),
        grid=(nkb,),
        in_specs=[
            pl.BlockSpec((Q, D), lambda k: (0, 0)),
            pl.BlockSpec((D, KB), lambda k: (0, k)),
            pl.BlockSpec((Q, 1), lambda k: (0, 0)),
            pl.BlockSpec((1, KB), lambda k: (0, k)),
        ],
        out_specs=[
            pl.BlockSpec((Q, TOPK), lambda k: (0, 0)),
            pl.BlockSpec((Q, TOPK), lambda k: (0, 0)),
        ],
        out_shape=[
            jax.ShapeDtypeStruct((Q, TOPK), jnp.float32),
            jax.ShapeDtypeStruct((Q, TOPK), jnp.int32),
        ],
        scratch_shapes=[
            pltpu.VMEM((Q, KB), jnp.float32),
            pltpu.VMEM((Q, TOPK), jnp.float32),
            pltpu.VMEM((Q, TOPK), jnp.float32),
            pltpu.VMEM((Q, TOPK), jnp.float32),
            pltpu.VMEM((Q, TOPK), jnp.float32),
        ],
    )(queries, keys_t, q2, k2p)
    return (vals, idx)
